# Initial kernel scaffold; baseline (speedup 1.0000x reference)
import jax
import jax.numpy as jnp
from jax.experimental import pallas as pl


def kernel(edge_index, features, W1, b1, W2, b2, W3, b3, att_W):
    raise NotImplementedError("write your pallas kernel here")


# speedup vs baseline: 19.3468x; 19.3468x over previous
import functools
import math

import jax
import jax.numpy as jnp
from jax import lax
from jax.experimental import pallas as pl
from jax.experimental.pallas import tpu as pltpu
from jax.experimental.pallas import tpu_sc as plsc

_CH = 128
_K = 4
_LANES = 16
_NTILES = 32


def _sc_mesh():
    return plsc.VectorSubcoreMesh(core_axis_name="c", subcore_axis_name="s")


def _make_deg(n_acc, nch):
    zch = n_acc // (16 * _CH)

    @functools.partial(
        pl.kernel,
        out_type=jax.ShapeDtypeStruct((2, n_acc, _LANES), jnp.float32),
        mesh=_sc_mesh(),
        compiler_params=pltpu.CompilerParams(use_tc_tiling_on_sc=False),
        scratch_types=[
            pltpu.VMEM((nch, _CH), jnp.int32),
            pltpu.VMEM((_CH, _LANES), jnp.float32),
            pltpu.VMEM_SHARED((n_acc, _LANES), jnp.float32),
            pltpu.SemaphoreType.DMA,
        ],
    )
    def deg_kernel(dst_hbm, out_hbm, dst_i, ones_v, acc_sh, sem):
        c = lax.axis_index("c")
        s = lax.axis_index("s")
        wid = s * 2 + c
        pltpu.sync_copy(dst_hbm.at[wid], dst_i)

        def fill(val):
            def body(i, carry):
                ones_v[i, :] = jnp.full((_LANES,), val, jnp.float32)
                return carry
            lax.fori_loop(0, _CH, body, 0)

        fill(0.0)

        def zacc(i, carry):
            pltpu.sync_copy(ones_v, acc_sh.at[pl.ds((s * zch + i) * _CH, _CH)])
            return carry
        lax.fori_loop(0, zch, zacc, 0)
        fill(1.0)
        plsc.subcore_barrier()

        def body(i, carry):
            for b in range(_K):
                pltpu.async_copy(ones_v, acc_sh.at[dst_i.at[i * _K + b]],
                                 sem, add=True)
            for b in range(_K):
                pltpu.make_async_copy(
                    ones_v, acc_sh.at[dst_i.at[i * _K + b]], sem).wait()
            return carry
        lax.fori_loop(0, nch // _K, body, 0)
        plsc.subcore_barrier()

        rpt = n_acc // 16
        pltpu.sync_copy(acc_sh.at[pl.ds(s * rpt, rpt)],
                        out_hbm.at[c].at[pl.ds(s * rpt, rpt)])

    return deg_kernel


def _make_prop(n_acc, nch, f):
    zch = n_acc // (16 * _CH)
    nb = nch // _K

    @functools.partial(
        pl.kernel,
        out_type=jax.ShapeDtypeStruct((2, n_acc, f), jnp.float32),
        mesh=_sc_mesh(),
        compiler_params=pltpu.CompilerParams(use_tc_tiling_on_sc=False),
        scratch_types=[
            pltpu.VMEM((nch, _CH), jnp.int32),
            pltpu.VMEM((nch, _CH), jnp.int32),
            pltpu.VMEM((_K * _CH, f), jnp.float32),
            pltpu.VMEM((_K * _CH, f), jnp.float32),
            pltpu.VMEM_SHARED((n_acc, f), jnp.float32),
            pltpu.SemaphoreType.DMA,
            pltpu.SemaphoreType.DMA,
            pltpu.SemaphoreType.DMA,
            pltpu.SemaphoreType.DMA,
        ],
    )
    def prop_kernel(h_hbm, src_hbm, dst_hbm, out_hbm,
                    src_i, dst_i, rows0, rows1, acc_sh, g0, g1, s0, s1):
        c = lax.axis_index("c")
        s = lax.axis_index("s")
        wid = s * 2 + c
        pltpu.sync_copy(src_hbm.at[wid], src_i)
        pltpu.sync_copy(dst_hbm.at[wid], dst_i)

        def zrow(i, carry):
            for j in range(f // _LANES):
                rows0[i, pl.ds(j * _LANES, _LANES)] = jnp.zeros(
                    (_LANES,), jnp.float32)
            return carry
        lax.fori_loop(0, _CH, zrow, 0)

        def zacc(i, carry):
            pltpu.sync_copy(rows0.at[pl.ds(0, _CH)],
                            acc_sh.at[pl.ds((s * zch + i) * _CH, _CH)])
            return carry
        lax.fori_loop(0, zch, zacc, 0)
        plsc.subcore_barrier()

        def fire(t, rows, sem):
            for b in range(_K):
                pltpu.async_copy(h_hbm.at[src_i.at[t * _K + b]],
                                 rows.at[pl.ds(b * _CH, _CH)], sem)

        def wait_rows(rows, sem):
            for b in range(_K):
                pltpu.make_async_copy(h_hbm.at[src_i.at[0]],
                                      rows.at[pl.ds(b * _CH, _CH)], sem).wait()

        def scat(t, rows, sem):
            for b in range(_K):
                pltpu.async_copy(rows.at[pl.ds(b * _CH, _CH)],
                                 acc_sh.at[dst_i.at[t * _K + b]], sem,
                                 add=True)

        def wait_scat(rows, sem):
            for b in range(_K):
                pltpu.make_async_copy(rows.at[pl.ds(b * _CH, _CH)],
                                      acc_sh.at[dst_i.at[0]], sem).wait()

        fire(0, rows0, g0)

        def body(i, carry):
            t = 2 * i
            fire(t + 1, rows1, g1)
            wait_rows(rows0, g0)
            scat(t, rows0, s0)
            wait_scat(rows0, s0)

            @pl.when(t + 2 < nb)
            def _():
                fire(t + 2, rows0, g0)

            wait_rows(rows1, g1)
            scat(t + 1, rows1, s1)
            wait_scat(rows1, s1)
            return carry
        lax.fori_loop(0, nb // 2, body, 0)
        plsc.subcore_barrier()

        rpt = n_acc // 16
        pltpu.sync_copy(acc_sh.at[pl.ds(s * rpt, rpt)],
                        out_hbm.at[c].at[pl.ds(s * rpt, rpt)])

    return prop_kernel


def _tc_first(n, f1):
    def body(feat_ref, w_ref, degp_ref, h_ref, dis_ref):
        deg = degp_ref[0, 0:n, 0:1] + degp_ref[1, 0:n, 0:1] + 1.0
        dis = lax.rsqrt(deg)
        h = jnp.dot(feat_ref[...], w_ref[...],
                    precision=lax.Precision.HIGHEST,
                    preferred_element_type=jnp.float32)
        h_ref[...] = h * dis
        dis_ref[...] = dis

    return pl.pallas_call(
        body,
        out_shape=(jax.ShapeDtypeStruct((n, f1), jnp.float32),
                   jax.ShapeDtypeStruct((n, 1), jnp.float32)))


def _tc_mid(n, f_out):
    def body(p_ref, h_ref, dis_ref, b_ref, w_ref, o_ref):
        dis = dis_ref[...]
        prop = p_ref[0, 0:n, :] + p_ref[1, 0:n, :] + h_ref[...]
        x = jnp.maximum(prop * dis + b_ref[...], 0.0)
        o_ref[...] = jnp.dot(x, w_ref[...],
                             precision=lax.Precision.HIGHEST,
                             preferred_element_type=jnp.float32) * dis

    return pl.pallas_call(
        body, out_shape=jax.ShapeDtypeStruct((n, f_out), jnp.float32))


def _tc_final(n, f3):
    def body(p_ref, h_ref, dis_ref, b_ref, aw_ref, abs_ref, pool_ref):
        abstract = ((p_ref[0, 0:n, :] + p_ref[1, 0:n, :] + h_ref[...])
                    * dis_ref[...] + b_ref[...])
        abs_ref[...] = abstract
        gc = jnp.mean(jnp.dot(abstract, aw_ref[...],
                              precision=lax.Precision.HIGHEST,
                              preferred_element_type=jnp.float32),
                      axis=0, keepdims=True)
        tg = jnp.tanh(gc)
        scores = jax.nn.sigmoid(jnp.sum(abstract * tg, axis=1, keepdims=True))
        pool_ref[...] = jnp.sum(abstract * scores, axis=0, keepdims=True)

    return pl.pallas_call(
        body,
        out_shape=(jax.ShapeDtypeStruct((n, f3), jnp.float32),
                   jax.ShapeDtypeStruct((1, f3), jnp.float32)))


def kernel(edge_index, features, W1, b1, W2, b2, W3, b3, att_W):
    n, _ = features.shape
    e = edge_index.shape[1]
    f1, f2, f3 = W1.shape[1], W2.shape[1], W3.shape[1]

    quantum = _CH * 2 * _K
    ept = math.ceil(e / (_NTILES * quantum)) * quantum
    e_pad = ept * _NTILES
    nch = ept // _CH
    n_acc = math.ceil((n + 1) / (16 * _CH)) * (16 * _CH)

    src = jnp.concatenate(
        [edge_index[0], jnp.zeros((e_pad - e,), jnp.int32)])
    dst = jnp.concatenate(
        [edge_index[1], jnp.full((e_pad - e,), n, jnp.int32)])
    src = src.reshape(_NTILES, nch, _CH)
    dst = dst.reshape(_NTILES, nch, _CH)

    degp = _make_deg(n_acc, nch)(dst)
    h1s, dis = _tc_first(n, f1)(features, W1, degp)
    q = _make_prop(n_acc, nch, f1)(h1s, src, dst)
    h2s = _tc_mid(n, f2)(q, h1s, dis, b1.reshape(1, f1), W2)
    r = _make_prop(n_acc, nch, f2)(h2s, src, dst)
    h3s = _tc_mid(n, f3)(r, h2s, dis, b2.reshape(1, f2), W3)
    sp = _make_prop(n_acc, nch, f3)(h3s, src, dst)
    abstract, pooled = _tc_final(n, f3)(sp, h3s, dis,
                                        b3.reshape(1, f3), att_W)
    return abstract, pooled.reshape(f3, 1)


# speedup vs baseline: 36.8830x; 1.9064x over previous
import functools
import math

import jax
import jax.numpy as jnp
from jax import lax
from jax.experimental import pallas as pl
from jax.experimental.pallas import tpu as pltpu
from jax.experimental.pallas import tpu_sc as plsc

_CH = 128
_K = 2
_LANES = 16
_NTILES = 32


def _sc_mesh():
    return plsc.VectorSubcoreMesh(core_axis_name="c", subcore_axis_name="s")


def _make_deg(n_acc, nch):
    zch = n_acc // (16 * _CH)

    @functools.partial(
        pl.kernel,
        out_type=jax.ShapeDtypeStruct((2, n_acc, _LANES), jnp.float32),
        mesh=_sc_mesh(),
        compiler_params=pltpu.CompilerParams(use_tc_tiling_on_sc=False),
        scratch_types=[
            pltpu.VMEM((nch, _CH), jnp.int32),
            pltpu.VMEM((_CH, _LANES), jnp.float32),
            pltpu.VMEM_SHARED((n_acc, _LANES), jnp.float32),
            pltpu.SemaphoreType.DMA,
        ],
    )
    def deg_kernel(dst_hbm, out_hbm, dst_i, ones_v, acc_sh, sem):
        c = lax.axis_index("c")
        s = lax.axis_index("s")
        wid = s * 2 + c
        pltpu.sync_copy(dst_hbm.at[wid], dst_i)

        def fill(val):
            def body(i, carry):
                ones_v[i, :] = jnp.full((_LANES,), val, jnp.float32)
                return carry
            lax.fori_loop(0, _CH, body, 0)

        fill(0.0)

        def zacc(i, carry):
            pltpu.sync_copy(ones_v, acc_sh.at[pl.ds((s * zch + i) * _CH, _CH)])
            return carry
        lax.fori_loop(0, zch, zacc, 0)
        fill(1.0)
        plsc.subcore_barrier()

        def body(i, carry):
            for b in range(_K):
                pltpu.async_copy(ones_v, acc_sh.at[dst_i.at[i * _K + b]],
                                 sem, add=True)
            for b in range(_K):
                pltpu.make_async_copy(
                    ones_v, acc_sh.at[dst_i.at[i * _K + b]], sem).wait()
            return carry
        lax.fori_loop(0, nch // _K, body, 0)
        plsc.subcore_barrier()

        rpt = n_acc // 16
        pltpu.sync_copy(acc_sh.at[pl.ds(s * rpt, rpt)],
                        out_hbm.at[c].at[pl.ds(s * rpt, rpt)])

    return deg_kernel


def _make_prop(n_acc, nch, f):
    zch = n_acc // (16 * _CH)
    nb = nch // _K

    @functools.partial(
        pl.kernel,
        out_type=jax.ShapeDtypeStruct((2, n_acc, f), jnp.float32),
        mesh=_sc_mesh(),
        compiler_params=pltpu.CompilerParams(use_tc_tiling_on_sc=False),
        scratch_types=[
            pltpu.VMEM((nch, _CH), jnp.int32),
            pltpu.VMEM((_K, _CH), jnp.int32),
            pltpu.VMEM((_K, _CH), jnp.int32),
            pltpu.VMEM((_K * _CH, f), jnp.float32),
            pltpu.VMEM((_K * _CH, f), jnp.float32),
            pltpu.VMEM_SHARED((n_acc, f), jnp.float32),
            pltpu.VMEM_SHARED((n_acc, f), jnp.float32),
            pltpu.SemaphoreType.DMA,
            pltpu.SemaphoreType.DMA,
            pltpu.SemaphoreType.DMA,
            pltpu.SemaphoreType.DMA,
            pltpu.SemaphoreType.DMA,
        ],
    )
    def prop_kernel(h_hbm, src_hbm, dst_hbm, out_hbm,
                    src_i, di0, di1, rows0, rows1, acc_sh, tab_sh,
                    g0, g1, s0, s1, isem):
        c = lax.axis_index("c")
        s = lax.axis_index("s")
        wid = s * 2 + c
        pltpu.sync_copy(src_hbm.at[wid], src_i)
        rpt = n_acc // 16
        pltpu.sync_copy(h_hbm.at[pl.ds(s * rpt, rpt)],
                        tab_sh.at[pl.ds(s * rpt, rpt)])

        def zrow(i, carry):
            for j in range(f // _LANES):
                rows0[i, pl.ds(j * _LANES, _LANES)] = jnp.zeros(
                    (_LANES,), jnp.float32)
            return carry
        lax.fori_loop(0, _CH, zrow, 0)

        def zacc(i, carry):
            pltpu.sync_copy(rows0.at[pl.ds(0, _CH)],
                            acc_sh.at[pl.ds((s * zch + i) * _CH, _CH)])
            return carry
        lax.fori_loop(0, zch, zacc, 0)
        plsc.subcore_barrier()

        def idx_load(t, di):
            pltpu.sync_copy(dst_hbm.at[wid].at[pl.ds(t * _K, _K)], di)

        def fire(t, rows, sem):
            for b in range(_K):
                pltpu.async_copy(tab_sh.at[src_i.at[t * _K + b]],
                                 rows.at[pl.ds(b * _CH, _CH)], sem)

        def wait_rows(rows, sem):
            for b in range(_K):
                pltpu.make_async_copy(tab_sh.at[src_i.at[0]],
                                      rows.at[pl.ds(b * _CH, _CH)],
                                      sem).wait()

        def scat(t, rows, di, sem):
            for b in range(_K):
                pltpu.async_copy(rows.at[pl.ds(b * _CH, _CH)],
                                 acc_sh.at[di.at[b]], sem, add=True)

        def wait_scat(rows, di, sem):
            for b in range(_K):
                pltpu.make_async_copy(rows.at[pl.ds(b * _CH, _CH)],
                                      acc_sh.at[di.at[b]], sem).wait()

        idx_load(0, di0)
        fire(0, rows0, g0)
        fire(1, rows1, g1)

        def body(i, carry):
            t = 2 * i
            pltpu.async_copy(dst_hbm.at[wid].at[pl.ds((t + 1) * _K, _K)],
                             di1, isem)
            wait_rows(rows0, g0)
            scat(t, rows0, di0, s0)
            wait_scat(rows0, di0, s0)

            @pl.when(t + 2 < nb)
            def _():
                fire(t + 2, rows0, g0)

            pltpu.make_async_copy(dst_hbm.at[wid].at[pl.ds(0, _K)],
                                  di1, isem).wait()
            wait_rows(rows1, g1)
            scat(t + 1, rows1, di1, s1)
            wait_scat(rows1, di1, s1)

            @pl.when(t + 3 < nb)
            def _():
                fire(t + 3, rows1, g1)

            @pl.when(t + 2 < nb)
            def _():
                idx_load(t + 2, di0)
            return carry
        lax.fori_loop(0, nb // 2, body, 0)
        plsc.subcore_barrier()

        pltpu.sync_copy(acc_sh.at[pl.ds(s * rpt, rpt)],
                        out_hbm.at[c].at[pl.ds(s * rpt, rpt)])

    return prop_kernel


def _tc_first(n, n_acc, f1):
    def body(feat_ref, w_ref, degp_ref, h_ref, dis_ref):
        deg = degp_ref[0, 0:n, 0:1] + degp_ref[1, 0:n, 0:1] + 1.0
        dis = lax.rsqrt(deg)
        h = jnp.dot(feat_ref[...], w_ref[...],
                    precision=lax.Precision.HIGHEST,
                    preferred_element_type=jnp.float32)
        h_ref[0:n, :] = h * dis
        h_ref[n:n_acc, :] = jnp.zeros((n_acc - n, h.shape[1]), jnp.float32)
        dis_ref[...] = dis

    return pl.pallas_call(
        body,
        out_shape=(jax.ShapeDtypeStruct((n_acc, f1), jnp.float32),
                   jax.ShapeDtypeStruct((n, 1), jnp.float32)))


def _tc_mid(n, n_acc, f_out):
    def body(p_ref, h_ref, dis_ref, b_ref, w_ref, o_ref):
        dis = dis_ref[...]
        prop = p_ref[0, 0:n, :] + p_ref[1, 0:n, :] + h_ref[0:n, :]
        x = jnp.maximum(prop * dis + b_ref[...], 0.0)
        o_ref[0:n, :] = jnp.dot(x, w_ref[...],
                                precision=lax.Precision.HIGHEST,
                                preferred_element_type=jnp.float32) * dis
        o_ref[n:n_acc, :] = jnp.zeros((n_acc - n, w_ref.shape[1]),
                                      jnp.float32)

    return pl.pallas_call(
        body, out_shape=jax.ShapeDtypeStruct((n_acc, f_out), jnp.float32))


def _tc_final(n, f3):
    def body(p_ref, h_ref, dis_ref, b_ref, aw_ref, abs_ref, pool_ref):
        abstract = ((p_ref[0, 0:n, :] + p_ref[1, 0:n, :] + h_ref[0:n, :])
                    * dis_ref[...] + b_ref[...])
        abs_ref[...] = abstract
        gc = jnp.mean(jnp.dot(abstract, aw_ref[...],
                              precision=lax.Precision.HIGHEST,
                              preferred_element_type=jnp.float32),
                      axis=0, keepdims=True)
        tg = jnp.tanh(gc)
        scores = jax.nn.sigmoid(jnp.sum(abstract * tg, axis=1, keepdims=True))
        pool_ref[...] = jnp.sum(abstract * scores, axis=0, keepdims=True)

    return pl.pallas_call(
        body,
        out_shape=(jax.ShapeDtypeStruct((n, f3), jnp.float32),
                   jax.ShapeDtypeStruct((1, f3), jnp.float32)))


def kernel(edge_index, features, W1, b1, W2, b2, W3, b3, att_W):
    n, _ = features.shape
    e = edge_index.shape[1]
    f1, f2, f3 = W1.shape[1], W2.shape[1], W3.shape[1]

    quantum = _CH * 2 * _K
    ept = math.ceil(e / (_NTILES * quantum)) * quantum
    e_pad = ept * _NTILES
    nch = ept // _CH
    n_acc = math.ceil((n + 1) / (16 * _CH)) * (16 * _CH)

    src = jnp.concatenate(
        [edge_index[0], jnp.zeros((e_pad - e,), jnp.int32)])
    dst = jnp.concatenate(
        [edge_index[1], jnp.full((e_pad - e,), n, jnp.int32)])
    src = src.reshape(_NTILES, nch, _CH)
    dst = dst.reshape(_NTILES, nch, _CH)

    degp = _make_deg(n_acc, nch)(dst)
    h1s, dis = _tc_first(n, n_acc, f1)(features, W1, degp)
    q = _make_prop(n_acc, nch, f1)(h1s, src, dst)
    h2s = _tc_mid(n, n_acc, f2)(q, h1s, dis, b1.reshape(1, f1), W2)
    r = _make_prop(n_acc, nch, f2)(h2s, src, dst)
    h3s = _tc_mid(n, n_acc, f3)(r, h2s, dis, b2.reshape(1, f2), W3)
    sp = _make_prop(n_acc, nch, f3)(h3s, src, dst)
    abstract, pooled = _tc_final(n, f3)(sp, h3s, dis,
                                        b3.reshape(1, f3), att_W)
    return abstract, pooled.reshape(f3, 1)


# speedup vs baseline: 37.9713x; 1.0295x over previous
import functools
import math

import jax
import jax.numpy as jnp
from jax import lax
from jax.experimental import pallas as pl
from jax.experimental.pallas import tpu as pltpu
from jax.experimental.pallas import tpu_sc as plsc

_CH = 128
_K = 2
_LANES = 16
_NTILES = 32


def _sc_mesh():
    return plsc.VectorSubcoreMesh(core_axis_name="c", subcore_axis_name="s")


def _make_deg(n_acc, nch, k):
    zch = n_acc // (16 * _CH)

    @functools.partial(
        pl.kernel,
        out_type=jax.ShapeDtypeStruct((2, n_acc, _LANES), jnp.float32),
        mesh=_sc_mesh(),
        compiler_params=pltpu.CompilerParams(use_tc_tiling_on_sc=False),
        scratch_types=[
            pltpu.VMEM((nch, _CH), jnp.int32),
            pltpu.VMEM((_CH, _LANES), jnp.float32),
            pltpu.VMEM_SHARED((n_acc, _LANES), jnp.float32),
            pltpu.SemaphoreType.DMA,
        ],
    )
    def deg_kernel(dst_hbm, out_hbm, dst_i, ones_v, acc_sh, sem):
        c = lax.axis_index("c")
        s = lax.axis_index("s")
        wid = s * 2 + c
        pltpu.sync_copy(dst_hbm.at[wid], dst_i)

        def fill(val):
            def body(i, carry):
                ones_v[i, :] = jnp.full((_LANES,), val, jnp.float32)
                return carry
            lax.fori_loop(0, _CH, body, 0)

        fill(0.0)

        def zacc(i, carry):
            pltpu.sync_copy(ones_v, acc_sh.at[pl.ds((s * zch + i) * _CH, _CH)])
            return carry
        lax.fori_loop(0, zch, zacc, 0)
        fill(1.0)
        plsc.subcore_barrier()

        def body(i, carry):
            for b in range(k):
                pltpu.async_copy(ones_v, acc_sh.at[dst_i.at[i * k + b]],
                                 sem, add=True)
            for b in range(k):
                pltpu.make_async_copy(
                    ones_v, acc_sh.at[dst_i.at[i * k + b]], sem).wait()
            return carry
        lax.fori_loop(0, nch // k, body, 0)
        plsc.subcore_barrier()

        rpt = n_acc // 16
        pltpu.sync_copy(acc_sh.at[pl.ds(s * rpt, rpt)],
                        out_hbm.at[c].at[pl.ds(s * rpt, rpt)])

    return deg_kernel


def _make_prop(n_acc, nch, f, k):
    zch = n_acc // (16 * _CH)
    nb = nch // k

    @functools.partial(
        pl.kernel,
        out_type=jax.ShapeDtypeStruct((2, n_acc, f), jnp.float32),
        mesh=_sc_mesh(),
        compiler_params=pltpu.CompilerParams(use_tc_tiling_on_sc=False),
        scratch_types=[
            pltpu.VMEM((nch, _CH), jnp.int32),
            pltpu.VMEM((k, _CH), jnp.int32),
            pltpu.VMEM((k, _CH), jnp.int32),
            pltpu.VMEM((k * _CH, f), jnp.float32),
            pltpu.VMEM((k * _CH, f), jnp.float32),
            pltpu.VMEM_SHARED((n_acc, f), jnp.float32),
            pltpu.VMEM_SHARED((n_acc, f), jnp.float32),
            pltpu.SemaphoreType.DMA,
            pltpu.SemaphoreType.DMA,
            pltpu.SemaphoreType.DMA,
            pltpu.SemaphoreType.DMA,
            pltpu.SemaphoreType.DMA,
        ],
    )
    def prop_kernel(h_hbm, src_hbm, dst_hbm, out_hbm,
                    src_i, di0, di1, rows0, rows1, acc_sh, tab_sh,
                    g0, g1, s0, s1, isem):
        c = lax.axis_index("c")
        s = lax.axis_index("s")
        wid = s * 2 + c
        pltpu.sync_copy(src_hbm.at[wid], src_i)
        rpt = n_acc // 16
        pltpu.sync_copy(h_hbm.at[pl.ds(s * rpt, rpt)],
                        tab_sh.at[pl.ds(s * rpt, rpt)])

        def zrow(i, carry):
            for j in range(f // _LANES):
                rows0[i, pl.ds(j * _LANES, _LANES)] = jnp.zeros(
                    (_LANES,), jnp.float32)
            return carry
        lax.fori_loop(0, _CH, zrow, 0)

        def zacc(i, carry):
            pltpu.sync_copy(rows0.at[pl.ds(0, _CH)],
                            acc_sh.at[pl.ds((s * zch + i) * _CH, _CH)])
            return carry
        lax.fori_loop(0, zch, zacc, 0)
        plsc.subcore_barrier()

        def idx_load(t, di):
            pltpu.sync_copy(dst_hbm.at[wid].at[pl.ds(t * k, k)], di)

        def fire(t, rows, sem):
            for b in range(k):
                pltpu.async_copy(tab_sh.at[src_i.at[t * k + b]],
                                 rows.at[pl.ds(b * _CH, _CH)], sem)

        def wait_rows(rows, sem):
            for b in range(k):
                pltpu.make_async_copy(tab_sh.at[src_i.at[0]],
                                      rows.at[pl.ds(b * _CH, _CH)],
                                      sem).wait()

        def scat(t, rows, di, sem):
            for b in range(k):
                pltpu.async_copy(rows.at[pl.ds(b * _CH, _CH)],
                                 acc_sh.at[di.at[b]], sem, add=True)

        def wait_scat(rows, di, sem):
            for b in range(k):
                pltpu.make_async_copy(rows.at[pl.ds(b * _CH, _CH)],
                                      acc_sh.at[di.at[b]], sem).wait()

        idx_load(0, di0)
        fire(0, rows0, g0)
        fire(1, rows1, g1)

        def body(i, carry):
            t = 2 * i
            pltpu.async_copy(dst_hbm.at[wid].at[pl.ds((t + 1) * k, k)],
                             di1, isem)
            wait_rows(rows0, g0)
            scat(t, rows0, di0, s0)
            wait_scat(rows0, di0, s0)

            @pl.when(t + 2 < nb)
            def _():
                fire(t + 2, rows0, g0)

            pltpu.make_async_copy(dst_hbm.at[wid].at[pl.ds(0, k)],
                                  di1, isem).wait()
            wait_rows(rows1, g1)
            scat(t + 1, rows1, di1, s1)
            wait_scat(rows1, di1, s1)

            @pl.when(t + 3 < nb)
            def _():
                fire(t + 3, rows1, g1)

            @pl.when(t + 2 < nb)
            def _():
                idx_load(t + 2, di0)
            return carry
        lax.fori_loop(0, nb // 2, body, 0)
        plsc.subcore_barrier()

        pltpu.sync_copy(acc_sh.at[pl.ds(s * rpt, rpt)],
                        out_hbm.at[c].at[pl.ds(s * rpt, rpt)])

    return prop_kernel


def _tc_first(n, n_acc, f1):
    def body(feat_ref, w_ref, degp_ref, h_ref, dis_ref):
        deg = degp_ref[0, 0:n, 0:1] + degp_ref[1, 0:n, 0:1] + 1.0
        dis = lax.rsqrt(deg)
        h = jnp.dot(feat_ref[...], w_ref[...],
                    precision=lax.Precision.HIGHEST,
                    preferred_element_type=jnp.float32)
        h_ref[0:n, :] = h * dis
        h_ref[n:n_acc, :] = jnp.zeros((n_acc - n, h.shape[1]), jnp.float32)
        dis_ref[...] = dis

    return pl.pallas_call(
        body,
        out_shape=(jax.ShapeDtypeStruct((n_acc, f1), jnp.float32),
                   jax.ShapeDtypeStruct((n, 1), jnp.float32)))


def _tc_mid(n, n_acc, f_out):
    def body(p_ref, h_ref, dis_ref, b_ref, w_ref, o_ref):
        dis = dis_ref[...]
        prop = p_ref[0, 0:n, :] + p_ref[1, 0:n, :] + h_ref[0:n, :]
        x = jnp.maximum(prop * dis + b_ref[...], 0.0)
        o_ref[0:n, :] = jnp.dot(x, w_ref[...],
                                precision=lax.Precision.HIGHEST,
                                preferred_element_type=jnp.float32) * dis
        o_ref[n:n_acc, :] = jnp.zeros((n_acc - n, w_ref.shape[1]),
                                      jnp.float32)

    return pl.pallas_call(
        body, out_shape=jax.ShapeDtypeStruct((n_acc, f_out), jnp.float32))


def _tc_final(n, f3):
    def body(p_ref, h_ref, dis_ref, b_ref, aw_ref, abs_ref, pool_ref):
        abstract = ((p_ref[0, 0:n, :] + p_ref[1, 0:n, :] + h_ref[0:n, :])
                    * dis_ref[...] + b_ref[...])
        abs_ref[...] = abstract
        gc = jnp.mean(jnp.dot(abstract, aw_ref[...],
                              precision=lax.Precision.HIGHEST,
                              preferred_element_type=jnp.float32),
                      axis=0, keepdims=True)
        tg = jnp.tanh(gc)
        scores = jax.nn.sigmoid(jnp.sum(abstract * tg, axis=1, keepdims=True))
        pool_ref[...] = jnp.sum(abstract * scores, axis=0, keepdims=True)

    return pl.pallas_call(
        body,
        out_shape=(jax.ShapeDtypeStruct((n, f3), jnp.float32),
                   jax.ShapeDtypeStruct((1, f3), jnp.float32)))


def kernel(edge_index, features, W1, b1, W2, b2, W3, b3, att_W):
    n, _ = features.shape
    e = edge_index.shape[1]
    f1, f2, f3 = W1.shape[1], W2.shape[1], W3.shape[1]

    quantum = _CH * 2 * 4
    ept = math.ceil(e / (_NTILES * quantum)) * quantum
    e_pad = ept * _NTILES
    nch = ept // _CH
    n_acc = math.ceil((n + 1) / (16 * _CH)) * (16 * _CH)

    src = jnp.concatenate(
        [edge_index[0], jnp.zeros((e_pad - e,), jnp.int32)])
    dst = jnp.concatenate(
        [edge_index[1], jnp.full((e_pad - e,), n, jnp.int32)])
    src = src.reshape(_NTILES, nch, _CH)
    dst = dst.reshape(_NTILES, nch, _CH)

    degp = _make_deg(n_acc, nch, 4)(dst)
    h1s, dis = _tc_first(n, n_acc, f1)(features, W1, degp)
    q = _make_prop(n_acc, nch, f1, 2)(h1s, src, dst)
    h2s = _tc_mid(n, n_acc, f2)(q, h1s, dis, b1.reshape(1, f1), W2)
    r = _make_prop(n_acc, nch, f2, 4)(h2s, src, dst)
    h3s = _tc_mid(n, n_acc, f3)(r, h2s, dis, b2.reshape(1, f2), W3)
    sp = _make_prop(n_acc, nch, f3, 4)(h3s, src, dst)
    abstract, pooled = _tc_final(n, f3)(sp, h3s, dis,
                                        b3.reshape(1, f3), att_W)
    return abstract, pooled.reshape(f3, 1)


# speedup vs baseline: 39.6315x; 1.0437x over previous
import functools
import math

import jax
import jax.numpy as jnp
from jax import lax
from jax.experimental import pallas as pl
from jax.experimental.pallas import tpu as pltpu
from jax.experimental.pallas import tpu_sc as plsc

_CH = 128
_K = 2
_LANES = 16
_NTILES = 32


def _sc_mesh():
    return plsc.VectorSubcoreMesh(core_axis_name="c", subcore_axis_name="s")


def _make_deg(n_acc, nch, k):
    w = 8
    zch = n_acc // (16 * _CH)

    @functools.partial(
        pl.kernel,
        out_type=jax.ShapeDtypeStruct((2, n_acc, w), jnp.float32),
        mesh=_sc_mesh(),
        compiler_params=pltpu.CompilerParams(use_tc_tiling_on_sc=False),
        scratch_types=[
            pltpu.VMEM((nch, _CH), jnp.int32),
            pltpu.VMEM((_CH, w), jnp.float32),
            pltpu.VMEM((_CH, w), jnp.float32),
            pltpu.VMEM_SHARED((n_acc, w), jnp.float32),
            pltpu.SemaphoreType.DMA,
        ],
    )
    def deg_kernel(ones_hbm, dst_hbm, out_hbm, dst_i, ones_v, zero_v,
                   acc_sh, sem):
        c = lax.axis_index("c")
        s = lax.axis_index("s")
        wid = s * 2 + c
        pltpu.sync_copy(dst_hbm.at[wid], dst_i)
        pltpu.sync_copy(ones_hbm.at[0], zero_v)
        pltpu.sync_copy(ones_hbm.at[1], ones_v)

        def zacc(i, carry):
            pltpu.sync_copy(zero_v, acc_sh.at[pl.ds((s * zch + i) * _CH, _CH)])
            return carry
        lax.fori_loop(0, zch, zacc, 0)
        plsc.subcore_barrier()

        def body(i, carry):
            for b in range(k):
                pltpu.async_copy(ones_v, acc_sh.at[dst_i.at[i * k + b]],
                                 sem, add=True)
            for b in range(k):
                pltpu.make_async_copy(
                    ones_v, acc_sh.at[dst_i.at[i * k + b]], sem).wait()
            return carry
        lax.fori_loop(0, nch // k, body, 0)
        plsc.subcore_barrier()

        rpt = n_acc // 16
        pltpu.sync_copy(acc_sh.at[pl.ds(s * rpt, rpt)],
                        out_hbm.at[c].at[pl.ds(s * rpt, rpt)])

    return deg_kernel


def _make_prop(n_acc, nch, f, k):
    zch = n_acc // (16 * _CH)
    nb = nch // k

    @functools.partial(
        pl.kernel,
        out_type=jax.ShapeDtypeStruct((2, n_acc, f), jnp.float32),
        mesh=_sc_mesh(),
        compiler_params=pltpu.CompilerParams(use_tc_tiling_on_sc=False),
        scratch_types=[
            pltpu.VMEM((nch, _CH), jnp.int32),
            pltpu.VMEM((k, _CH), jnp.int32),
            pltpu.VMEM((k, _CH), jnp.int32),
            pltpu.VMEM((k * _CH, f), jnp.float32),
            pltpu.VMEM((k * _CH, f), jnp.float32),
            pltpu.VMEM_SHARED((n_acc, f), jnp.float32),
            pltpu.VMEM_SHARED((n_acc, f), jnp.float32),
            pltpu.SemaphoreType.DMA,
            pltpu.SemaphoreType.DMA,
            pltpu.SemaphoreType.DMA,
            pltpu.SemaphoreType.DMA,
            pltpu.SemaphoreType.DMA,
        ],
    )
    def prop_kernel(h_hbm, src_hbm, dst_hbm, out_hbm,
                    src_i, di0, di1, rows0, rows1, acc_sh, tab_sh,
                    g0, g1, s0, s1, isem):
        c = lax.axis_index("c")
        s = lax.axis_index("s")
        wid = s * 2 + c
        pltpu.sync_copy(src_hbm.at[wid], src_i)
        rpt = n_acc // 16
        pltpu.sync_copy(h_hbm.at[pl.ds(s * rpt, rpt)],
                        tab_sh.at[pl.ds(s * rpt, rpt)])

        def zrow(i, carry):
            for j in range(f // _LANES):
                rows0[i, pl.ds(j * _LANES, _LANES)] = jnp.zeros(
                    (_LANES,), jnp.float32)
            return carry
        lax.fori_loop(0, _CH, zrow, 0)

        def zacc(i, carry):
            pltpu.sync_copy(rows0.at[pl.ds(0, _CH)],
                            acc_sh.at[pl.ds((s * zch + i) * _CH, _CH)])
            return carry
        lax.fori_loop(0, zch, zacc, 0)
        plsc.subcore_barrier()

        def idx_load(t, di):
            pltpu.sync_copy(dst_hbm.at[wid].at[pl.ds(t * k, k)], di)

        def fire(t, rows, sem):
            for b in range(k):
                pltpu.async_copy(tab_sh.at[src_i.at[t * k + b]],
                                 rows.at[pl.ds(b * _CH, _CH)], sem)

        def wait_rows(rows, sem):
            for b in range(k):
                pltpu.make_async_copy(tab_sh.at[src_i.at[0]],
                                      rows.at[pl.ds(b * _CH, _CH)],
                                      sem).wait()

        def scat(t, rows, di, sem):
            for b in range(k):
                pltpu.async_copy(rows.at[pl.ds(b * _CH, _CH)],
                                 acc_sh.at[di.at[b]], sem, add=True)

        def wait_scat(rows, di, sem):
            for b in range(k):
                pltpu.make_async_copy(rows.at[pl.ds(b * _CH, _CH)],
                                      acc_sh.at[di.at[b]], sem).wait()

        idx_load(0, di0)
        fire(0, rows0, g0)
        fire(1, rows1, g1)

        def body(i, carry):
            t = 2 * i
            pltpu.async_copy(dst_hbm.at[wid].at[pl.ds((t + 1) * k, k)],
                             di1, isem)
            wait_rows(rows0, g0)
            scat(t, rows0, di0, s0)
            wait_scat(rows0, di0, s0)

            @pl.when(t + 2 < nb)
            def _():
                fire(t + 2, rows0, g0)

            pltpu.make_async_copy(dst_hbm.at[wid].at[pl.ds(0, k)],
                                  di1, isem).wait()
            wait_rows(rows1, g1)
            scat(t + 1, rows1, di1, s1)
            wait_scat(rows1, di1, s1)

            @pl.when(t + 3 < nb)
            def _():
                fire(t + 3, rows1, g1)

            @pl.when(t + 2 < nb)
            def _():
                idx_load(t + 2, di0)
            return carry
        lax.fori_loop(0, nb // 2, body, 0)
        plsc.subcore_barrier()

        pltpu.sync_copy(acc_sh.at[pl.ds(s * rpt, rpt)],
                        out_hbm.at[c].at[pl.ds(s * rpt, rpt)])

    return prop_kernel


def _tc_first(n, n_acc, f1):
    def body(feat_ref, w_ref, degp_ref, h_ref, dis_ref):
        deg = degp_ref[0, 0:n, 0:1] + degp_ref[1, 0:n, 0:1] + 1.0
        dis = lax.rsqrt(deg)
        h = jnp.dot(feat_ref[...], w_ref[...],
                    precision=lax.Precision.DEFAULT,
                    preferred_element_type=jnp.float32)
        h_ref[0:n, :] = h * dis
        h_ref[n:n_acc, :] = jnp.zeros((n_acc - n, h.shape[1]), jnp.float32)
        dis_ref[...] = dis

    return pl.pallas_call(
        body,
        out_shape=(jax.ShapeDtypeStruct((n_acc, f1), jnp.float32),
                   jax.ShapeDtypeStruct((n, 1), jnp.float32)))


def _tc_mid(n, n_acc, f_out):
    def body(p_ref, h_ref, dis_ref, b_ref, w_ref, o_ref):
        dis = dis_ref[...]
        prop = p_ref[0, 0:n, :] + p_ref[1, 0:n, :] + h_ref[0:n, :]
        x = jnp.maximum(prop * dis + b_ref[...], 0.0)
        o_ref[0:n, :] = jnp.dot(x, w_ref[...],
                                precision=lax.Precision.DEFAULT,
                                preferred_element_type=jnp.float32) * dis
        o_ref[n:n_acc, :] = jnp.zeros((n_acc - n, w_ref.shape[1]),
                                      jnp.float32)

    return pl.pallas_call(
        body, out_shape=jax.ShapeDtypeStruct((n_acc, f_out), jnp.float32))


def _tc_final(n, f3):
    def body(p_ref, h_ref, dis_ref, b_ref, aw_ref, abs_ref, pool_ref):
        abstract = ((p_ref[0, 0:n, :] + p_ref[1, 0:n, :] + h_ref[0:n, :])
                    * dis_ref[...] + b_ref[...])
        abs_ref[...] = abstract
        gc = jnp.mean(jnp.dot(abstract, aw_ref[...],
                              precision=lax.Precision.DEFAULT,
                              preferred_element_type=jnp.float32),
                      axis=0, keepdims=True)
        tg = jnp.tanh(gc)
        scores = jax.nn.sigmoid(jnp.sum(abstract * tg, axis=1, keepdims=True))
        pool_ref[...] = jnp.sum(abstract * scores, axis=0, keepdims=True)

    return pl.pallas_call(
        body,
        out_shape=(jax.ShapeDtypeStruct((n, f3), jnp.float32),
                   jax.ShapeDtypeStruct((1, f3), jnp.float32)))


def kernel(edge_index, features, W1, b1, W2, b2, W3, b3, att_W):
    n, _ = features.shape
    e = edge_index.shape[1]
    f1, f2, f3 = W1.shape[1], W2.shape[1], W3.shape[1]

    quantum = _CH * 2 * 4
    ept = math.ceil(e / (_NTILES * quantum)) * quantum
    e_pad = ept * _NTILES
    nch = ept // _CH
    n_acc = math.ceil((n + 1) / (16 * _CH)) * (16 * _CH)

    src = jnp.concatenate(
        [edge_index[0], jnp.zeros((e_pad - e,), jnp.int32)])
    dst = jnp.concatenate(
        [edge_index[1], jnp.full((e_pad - e,), n, jnp.int32)])
    src = src.reshape(_NTILES, nch, _CH)
    dst = dst.reshape(_NTILES, nch, _CH)

    oz = jnp.stack([jnp.zeros((_CH, 8), jnp.float32),
                jnp.ones((_CH, 8), jnp.float32)])
    degp = _make_deg(n_acc, nch, 4)(oz, dst)
    h1s, dis = _tc_first(n, n_acc, f1)(features, W1, degp)
    q = _make_prop(n_acc, nch, f1, 2)(h1s, src, dst)
    h2s = _tc_mid(n, n_acc, f2)(q, h1s, dis, b1.reshape(1, f1), W2)
    r = _make_prop(n_acc, nch, f2, 4)(h2s, src, dst)
    h3s = _tc_mid(n, n_acc, f3)(r, h2s, dis, b2.reshape(1, f2), W3)
    sp = _make_prop(n_acc, nch, f3, 4)(h3s, src, dst)
    abstract, pooled = _tc_final(n, f3)(sp, h3s, dis,
                                        b3.reshape(1, f3), att_W)
    return abstract, pooled.reshape(f3, 1)


# speedup vs baseline: 41.3118x; 1.0424x over previous
import functools
import math

import jax
import jax.numpy as jnp
from jax import lax
from jax.experimental import pallas as pl
from jax.experimental.pallas import tpu as pltpu
from jax.experimental.pallas import tpu_sc as plsc

_CH = 128
_ZCH = 128
_K = 2
_LANES = 16
_NTILES = 32


def _sc_mesh():
    return plsc.VectorSubcoreMesh(core_axis_name="c", subcore_axis_name="s")


def _make_deg(n_acc, nch, ch, k):
    w = 8
    zch = n_acc // (16 * _ZCH)

    @functools.partial(
        pl.kernel,
        out_type=jax.ShapeDtypeStruct((2, n_acc, w), jnp.float32),
        mesh=_sc_mesh(),
        compiler_params=pltpu.CompilerParams(use_tc_tiling_on_sc=False),
        scratch_types=[
            pltpu.VMEM((nch, ch), jnp.int32),
            pltpu.VMEM((ch, w), jnp.float32),
            pltpu.VMEM((_ZCH, w), jnp.float32),
            pltpu.VMEM_SHARED((n_acc, w), jnp.float32),
            pltpu.SemaphoreType.DMA,
        ],
    )
    def deg_kernel(zeros_hbm, ones_hbm, dst_hbm, out_hbm, dst_i, ones_v,
                   zero_v, acc_sh, sem):
        c = lax.axis_index("c")
        s = lax.axis_index("s")
        wid = s * 2 + c
        pltpu.sync_copy(dst_hbm.at[wid], dst_i)
        pltpu.sync_copy(zeros_hbm, zero_v)
        pltpu.sync_copy(ones_hbm, ones_v)

        def zacc(i, carry):
            pltpu.sync_copy(zero_v,
                            acc_sh.at[pl.ds((s * zch + i) * _ZCH, _ZCH)])
            return carry
        lax.fori_loop(0, zch, zacc, 0)
        plsc.subcore_barrier()

        def body(i, carry):
            for b in range(k):
                pltpu.async_copy(ones_v, acc_sh.at[dst_i.at[i * k + b]],
                                 sem, add=True)
            for b in range(k):
                pltpu.make_async_copy(
                    ones_v, acc_sh.at[dst_i.at[i * k + b]], sem).wait()
            return carry
        lax.fori_loop(0, nch // k, body, 0)
        plsc.subcore_barrier()

        rpt = n_acc // 16
        pltpu.sync_copy(acc_sh.at[pl.ds(s * rpt, rpt)],
                        out_hbm.at[c].at[pl.ds(s * rpt, rpt)])

    return deg_kernel


def _make_prop(n_acc, nch, ch, f, k):
    zch = n_acc // (16 * _ZCH)
    nb = nch // k

    @functools.partial(
        pl.kernel,
        out_type=jax.ShapeDtypeStruct((2, n_acc, f), jnp.float32),
        mesh=_sc_mesh(),
        compiler_params=pltpu.CompilerParams(use_tc_tiling_on_sc=False),
        scratch_types=[
            pltpu.VMEM((nch, ch), jnp.int32),
            pltpu.VMEM((k, ch), jnp.int32),
            pltpu.VMEM((k, ch), jnp.int32),
            pltpu.VMEM((k * ch, f), jnp.float32),
            pltpu.VMEM((k * ch, f), jnp.float32),
            pltpu.VMEM_SHARED((n_acc, f), jnp.float32),
            pltpu.VMEM_SHARED((n_acc, f), jnp.float32),
            pltpu.SemaphoreType.DMA,
            pltpu.SemaphoreType.DMA,
            pltpu.SemaphoreType.DMA,
            pltpu.SemaphoreType.DMA,
            pltpu.SemaphoreType.DMA,
        ],
    )
    def prop_kernel(h_hbm, src_hbm, dst_hbm, out_hbm,
                    src_i, di0, di1, rows0, rows1, acc_sh, tab_sh,
                    g0, g1, s0, s1, isem):
        c = lax.axis_index("c")
        s = lax.axis_index("s")
        wid = s * 2 + c
        pltpu.sync_copy(src_hbm.at[wid], src_i)
        rpt = n_acc // 16
        pltpu.sync_copy(h_hbm.at[pl.ds(s * rpt, rpt)],
                        tab_sh.at[pl.ds(s * rpt, rpt)])

        def zrow(i, carry):
            for j in range(f // _LANES):
                rows0[i, pl.ds(j * _LANES, _LANES)] = jnp.zeros(
                    (_LANES,), jnp.float32)
            return carry
        lax.fori_loop(0, _ZCH, zrow, 0)

        def zacc(i, carry):
            pltpu.sync_copy(rows0.at[pl.ds(0, _ZCH)],
                            acc_sh.at[pl.ds((s * zch + i) * _ZCH, _ZCH)])
            return carry
        lax.fori_loop(0, zch, zacc, 0)
        plsc.subcore_barrier()

        def idx_load(t, di):
            pltpu.sync_copy(dst_hbm.at[wid].at[pl.ds(t * k, k)], di)

        def fire(t, rows, sem):
            for b in range(k):
                pltpu.async_copy(tab_sh.at[src_i.at[t * k + b]],
                                 rows.at[pl.ds(b * ch, ch)], sem)

        def wait_rows(rows, sem):
            for b in range(k):
                pltpu.make_async_copy(tab_sh.at[src_i.at[0]],
                                      rows.at[pl.ds(b * ch, ch)],
                                      sem).wait()

        def scat(t, rows, di, sem):
            for b in range(k):
                pltpu.async_copy(rows.at[pl.ds(b * ch, ch)],
                                 acc_sh.at[di.at[b]], sem, add=True)

        def wait_scat(rows, di, sem):
            for b in range(k):
                pltpu.make_async_copy(rows.at[pl.ds(b * ch, ch)],
                                      acc_sh.at[di.at[b]], sem).wait()

        idx_load(0, di0)
        fire(0, rows0, g0)
        fire(1, rows1, g1)

        def body(i, carry):
            t = 2 * i
            pltpu.async_copy(dst_hbm.at[wid].at[pl.ds((t + 1) * k, k)],
                             di1, isem)
            wait_rows(rows0, g0)
            scat(t, rows0, di0, s0)
            wait_scat(rows0, di0, s0)

            @pl.when(t + 2 < nb)
            def _():
                fire(t + 2, rows0, g0)

            pltpu.make_async_copy(dst_hbm.at[wid].at[pl.ds(0, k)],
                                  di1, isem).wait()
            wait_rows(rows1, g1)
            scat(t + 1, rows1, di1, s1)
            wait_scat(rows1, di1, s1)

            @pl.when(t + 3 < nb)
            def _():
                fire(t + 3, rows1, g1)

            @pl.when(t + 2 < nb)
            def _():
                idx_load(t + 2, di0)
            return carry
        lax.fori_loop(0, nb // 2, body, 0)
        plsc.subcore_barrier()

        pltpu.sync_copy(acc_sh.at[pl.ds(s * rpt, rpt)],
                        out_hbm.at[c].at[pl.ds(s * rpt, rpt)])

    return prop_kernel


def _tc_first(n, n_acc, f1):
    def body(feat_ref, w_ref, degp_ref, h_ref, dis_ref):
        deg = degp_ref[0, 0:n, 0:1] + degp_ref[1, 0:n, 0:1] + 1.0
        dis = lax.rsqrt(deg)
        h = jnp.dot(feat_ref[...], w_ref[...],
                    precision=lax.Precision.DEFAULT,
                    preferred_element_type=jnp.float32)
        h_ref[0:n, :] = h * dis
        h_ref[n:n_acc, :] = jnp.zeros((n_acc - n, h.shape[1]), jnp.float32)
        dis_ref[...] = dis

    return pl.pallas_call(
        body,
        out_shape=(jax.ShapeDtypeStruct((n_acc, f1), jnp.float32),
                   jax.ShapeDtypeStruct((n, 1), jnp.float32)))


def _tc_mid(n, n_acc, f_out):
    def body(p_ref, h_ref, dis_ref, b_ref, w_ref, o_ref):
        dis = dis_ref[...]
        prop = p_ref[0, 0:n, :] + p_ref[1, 0:n, :] + h_ref[0:n, :]
        x = jnp.maximum(prop * dis + b_ref[...], 0.0)
        o_ref[0:n, :] = jnp.dot(x, w_ref[...],
                                precision=lax.Precision.DEFAULT,
                                preferred_element_type=jnp.float32) * dis
        o_ref[n:n_acc, :] = jnp.zeros((n_acc - n, w_ref.shape[1]),
                                      jnp.float32)

    return pl.pallas_call(
        body, out_shape=jax.ShapeDtypeStruct((n_acc, f_out), jnp.float32))


def _tc_final(n, f3):
    def body(p_ref, h_ref, dis_ref, b_ref, aw_ref, abs_ref, pool_ref):
        abstract = ((p_ref[0, 0:n, :] + p_ref[1, 0:n, :] + h_ref[0:n, :])
                    * dis_ref[...] + b_ref[...])
        abs_ref[...] = abstract
        gc = jnp.mean(jnp.dot(abstract, aw_ref[...],
                              precision=lax.Precision.DEFAULT,
                              preferred_element_type=jnp.float32),
                      axis=0, keepdims=True)
        tg = jnp.tanh(gc)
        scores = jax.nn.sigmoid(jnp.sum(abstract * tg, axis=1, keepdims=True))
        pool_ref[...] = jnp.sum(abstract * scores, axis=0, keepdims=True)

    return pl.pallas_call(
        body,
        out_shape=(jax.ShapeDtypeStruct((n, f3), jnp.float32),
                   jax.ShapeDtypeStruct((1, f3), jnp.float32)))


def kernel(edge_index, features, W1, b1, W2, b2, W3, b3, att_W):
    n, _ = features.shape
    e = edge_index.shape[1]
    f1, f2, f3 = W1.shape[1], W2.shape[1], W3.shape[1]

    def _pick_ch(edges):
        per_tile = edges // _NTILES
        if edges % _NTILES == 0:
            for c in range(_CH, 15, -1):
                if per_tile % (c * 8) == 0:
                    return c, per_tile // c, 0
        quantum = _CH * 8
        ept = math.ceil(edges / (_NTILES * quantum)) * quantum
        return _CH, ept // _CH, ept * _NTILES - edges

    ch, nch, pad = _pick_ch(e)
    n_acc = math.ceil((n + 1) / (16 * _ZCH)) * (16 * _ZCH)

    if pad:
        src = jnp.concatenate(
            [edge_index[0], jnp.zeros((pad,), jnp.int32)])
        dst = jnp.concatenate(
            [edge_index[1], jnp.full((pad,), n, jnp.int32)])
    else:
        src, dst = edge_index[0], edge_index[1]
    src = src.reshape(_NTILES, nch, ch)
    dst = dst.reshape(_NTILES, nch, ch)

    z8 = jnp.zeros((_ZCH, 8), jnp.float32)
    o8 = jnp.ones((ch, 8), jnp.float32)
    degp = _make_deg(n_acc, nch, ch, 4)(z8, o8, dst)
    h1s, dis = _tc_first(n, n_acc, f1)(features, W1, degp)
    q = _make_prop(n_acc, nch, ch, f1, 2)(h1s, src, dst)
    h2s = _tc_mid(n, n_acc, f2)(q, h1s, dis, b1.reshape(1, f1), W2)
    r = _make_prop(n_acc, nch, ch, f2, 4)(h2s, src, dst)
    h3s = _tc_mid(n, n_acc, f3)(r, h2s, dis, b2.reshape(1, f2), W3)
    sp = _make_prop(n_acc, nch, ch, f3, 4)(h3s, src, dst)
    abstract, pooled = _tc_final(n, f3)(sp, h3s, dis,
                                        b3.reshape(1, f3), att_W)
    return abstract, pooled.reshape(f3, 1)


# speedup vs baseline: 41.3235x; 1.0003x over previous
import functools
import math

import jax
import jax.numpy as jnp
from jax import lax
from jax.experimental import pallas as pl
from jax.experimental.pallas import tpu as pltpu
from jax.experimental.pallas import tpu_sc as plsc

_CH = 128
_ZCH = 128
_K = 2
_LANES = 16
_NTILES = 32


def _sc_mesh():
    return plsc.VectorSubcoreMesh(core_axis_name="c", subcore_axis_name="s")


def _make_deg(n_acc, nch, ch, k):
    w = 8
    zch = n_acc // (16 * _ZCH)

    @functools.partial(
        pl.kernel,
        out_type=jax.ShapeDtypeStruct((2, n_acc, w), jnp.float32),
        mesh=_sc_mesh(),
        compiler_params=pltpu.CompilerParams(use_tc_tiling_on_sc=False),
        scratch_types=[
            pltpu.VMEM((nch, ch), jnp.int32),
            pltpu.VMEM((ch, w), jnp.float32),
            pltpu.VMEM((_ZCH, w), jnp.float32),
            pltpu.VMEM_SHARED((n_acc, w), jnp.float32),
            pltpu.SemaphoreType.DMA,
        ],
    )
    def deg_kernel(zeros_hbm, ones_hbm, dst_hbm, out_hbm, dst_i, ones_v,
                   zero_v, acc_sh, sem):
        c = lax.axis_index("c")
        s = lax.axis_index("s")
        wid = s * 2 + c
        pltpu.sync_copy(dst_hbm.at[wid], dst_i)
        pltpu.sync_copy(zeros_hbm, zero_v)
        pltpu.sync_copy(ones_hbm, ones_v)

        def zacc(i, carry):
            pltpu.sync_copy(zero_v,
                            acc_sh.at[pl.ds((s * zch + i) * _ZCH, _ZCH)])
            return carry
        lax.fori_loop(0, zch, zacc, 0)
        plsc.subcore_barrier()

        def body(i, carry):
            for b in range(k):
                pltpu.async_copy(ones_v, acc_sh.at[dst_i.at[i * k + b]],
                                 sem, add=True)
            for b in range(k):
                pltpu.make_async_copy(
                    ones_v, acc_sh.at[dst_i.at[i * k + b]], sem).wait()
            return carry
        lax.fori_loop(0, nch // k, body, 0)
        plsc.subcore_barrier()

        rpt = n_acc // 16
        pltpu.sync_copy(acc_sh.at[pl.ds(s * rpt, rpt)],
                        out_hbm.at[c].at[pl.ds(s * rpt, rpt)])

    return deg_kernel


def _make_prop(n_acc, nch, ch, f, k):
    zch = n_acc // (16 * _ZCH)
    nb = nch // k

    @functools.partial(
        pl.kernel,
        out_type=jax.ShapeDtypeStruct((2, n_acc, f), jnp.float32),
        mesh=_sc_mesh(),
        compiler_params=pltpu.CompilerParams(use_tc_tiling_on_sc=False),
        scratch_types=[
            pltpu.VMEM((nch, ch), jnp.int32),
            pltpu.VMEM((k, ch), jnp.int32),
            pltpu.VMEM((k, ch), jnp.int32),
            pltpu.VMEM((k * ch, f), jnp.float32),
            pltpu.VMEM((k * ch, f), jnp.float32),
            pltpu.VMEM_SHARED((n_acc, f), jnp.float32),
            pltpu.VMEM_SHARED((n_acc, f), jnp.float32),
            pltpu.SemaphoreType.DMA,
            pltpu.SemaphoreType.DMA,
            pltpu.SemaphoreType.DMA,
            pltpu.SemaphoreType.DMA,
            pltpu.SemaphoreType.DMA,
        ],
    )
    def prop_kernel(h_hbm, src_hbm, dst_hbm, out_hbm,
                    src_i, di0, di1, rows0, rows1, acc_sh, tab_sh,
                    g0, g1, s0, s1, isem):
        c = lax.axis_index("c")
        s = lax.axis_index("s")
        wid = s * 2 + c
        pltpu.sync_copy(src_hbm.at[wid], src_i)
        rpt = n_acc // 16
        pltpu.sync_copy(h_hbm.at[pl.ds(s * rpt, rpt)],
                        tab_sh.at[pl.ds(s * rpt, rpt)])

        def zrow(i, carry):
            for j in range(f // _LANES):
                rows0[i, pl.ds(j * _LANES, _LANES)] = jnp.zeros(
                    (_LANES,), jnp.float32)
            return carry
        lax.fori_loop(0, _ZCH, zrow, 0)

        def zacc(i, carry):
            pltpu.sync_copy(rows0.at[pl.ds(0, _ZCH)],
                            acc_sh.at[pl.ds((s * zch + i) * _ZCH, _ZCH)])
            return carry
        lax.fori_loop(0, zch, zacc, 0)
        plsc.subcore_barrier()

        def idx_load(t, di):
            pltpu.sync_copy(dst_hbm.at[wid].at[pl.ds(t * k, k)], di)

        def fire(t, rows, sem):
            for b in range(k):
                pltpu.async_copy(tab_sh.at[src_i.at[t * k + b]],
                                 rows.at[pl.ds(b * ch, ch)], sem)

        def wait_rows(rows, sem):
            for b in range(k):
                pltpu.make_async_copy(tab_sh.at[src_i.at[0]],
                                      rows.at[pl.ds(b * ch, ch)],
                                      sem).wait()

        def scat(t, rows, di, sem):
            for b in range(k):
                pltpu.async_copy(rows.at[pl.ds(b * ch, ch)],
                                 acc_sh.at[di.at[b]], sem, add=True)

        def wait_scat(rows, di, sem):
            for b in range(k):
                pltpu.make_async_copy(rows.at[pl.ds(b * ch, ch)],
                                      acc_sh.at[di.at[b]], sem).wait()

        idx_load(0, di0)
        fire(0, rows0, g0)
        fire(1, rows1, g1)

        def body(i, carry):
            t = 2 * i
            pltpu.async_copy(dst_hbm.at[wid].at[pl.ds((t + 1) * k, k)],
                             di1, isem)
            wait_rows(rows0, g0)
            scat(t, rows0, di0, s0)
            wait_scat(rows0, di0, s0)

            @pl.when(t + 2 < nb)
            def _():
                fire(t + 2, rows0, g0)

            pltpu.make_async_copy(dst_hbm.at[wid].at[pl.ds(0, k)],
                                  di1, isem).wait()
            wait_rows(rows1, g1)
            scat(t + 1, rows1, di1, s1)
            wait_scat(rows1, di1, s1)

            @pl.when(t + 3 < nb)
            def _():
                fire(t + 3, rows1, g1)

            @pl.when(t + 2 < nb)
            def _():
                idx_load(t + 2, di0)
            return carry
        lax.fori_loop(0, nb // 2, body, 0)
        plsc.subcore_barrier()

        pltpu.sync_copy(acc_sh.at[pl.ds(s * rpt, rpt)],
                        out_hbm.at[c].at[pl.ds(s * rpt, rpt)])

    return prop_kernel


def _tc_mm(n, n_acc, f1):
    def body(feat_ref, w_ref, h_ref):
        h = jnp.dot(feat_ref[...], w_ref[...],
                    precision=lax.Precision.DEFAULT,
                    preferred_element_type=jnp.float32)
        h_ref[0:n, :] = h
        h_ref[n:n_acc, :] = jnp.zeros((n_acc - n, h.shape[1]), jnp.float32)

    return pl.pallas_call(
        body, out_shape=jax.ShapeDtypeStruct((n_acc, f1), jnp.float32))


def _tc_scale(n, n_acc, f1):
    def body(h_raw_ref, degp_ref, h_ref, dis_ref):
        deg = degp_ref[0, 0:n, 0:1] + degp_ref[1, 0:n, 0:1] + 1.0
        dis = lax.rsqrt(deg)
        h_ref[0:n, :] = h_raw_ref[0:n, :] * dis
        h_ref[n:n_acc, :] = jnp.zeros((n_acc - n, f1), jnp.float32)
        dis_ref[...] = dis

    return pl.pallas_call(
        body,
        out_shape=(jax.ShapeDtypeStruct((n_acc, f1), jnp.float32),
                   jax.ShapeDtypeStruct((n, 1), jnp.float32)))


def _tc_mid(n, n_acc, f_out):
    def body(p_ref, h_ref, dis_ref, b_ref, w_ref, o_ref):
        dis = dis_ref[...]
        prop = p_ref[0, 0:n, :] + p_ref[1, 0:n, :] + h_ref[0:n, :]
        x = jnp.maximum(prop * dis + b_ref[...], 0.0)
        o_ref[0:n, :] = jnp.dot(x, w_ref[...],
                                precision=lax.Precision.DEFAULT,
                                preferred_element_type=jnp.float32) * dis
        o_ref[n:n_acc, :] = jnp.zeros((n_acc - n, w_ref.shape[1]),
                                      jnp.float32)

    return pl.pallas_call(
        body, out_shape=jax.ShapeDtypeStruct((n_acc, f_out), jnp.float32))


def _tc_final(n, f3):
    def body(p_ref, h_ref, dis_ref, b_ref, aw_ref, abs_ref, pool_ref):
        abstract = ((p_ref[0, 0:n, :] + p_ref[1, 0:n, :] + h_ref[0:n, :])
                    * dis_ref[...] + b_ref[...])
        abs_ref[...] = abstract
        gc = jnp.mean(jnp.dot(abstract, aw_ref[...],
                              precision=lax.Precision.DEFAULT,
                              preferred_element_type=jnp.float32),
                      axis=0, keepdims=True)
        tg = jnp.tanh(gc)
        scores = jax.nn.sigmoid(jnp.sum(abstract * tg, axis=1, keepdims=True))
        pool_ref[...] = jnp.sum(abstract * scores, axis=0, keepdims=True)

    return pl.pallas_call(
        body,
        out_shape=(jax.ShapeDtypeStruct((n, f3), jnp.float32),
                   jax.ShapeDtypeStruct((1, f3), jnp.float32)))


def kernel(edge_index, features, W1, b1, W2, b2, W3, b3, att_W):
    n, _ = features.shape
    e = edge_index.shape[1]
    f1, f2, f3 = W1.shape[1], W2.shape[1], W3.shape[1]

    def _pick_ch(edges):
        per_tile = edges // _NTILES
        if edges % _NTILES == 0:
            for c in range(_CH, 15, -1):
                if per_tile % (c * 8) == 0:
                    return c, per_tile // c, 0
        quantum = _CH * 8
        ept = math.ceil(edges / (_NTILES * quantum)) * quantum
        return _CH, ept // _CH, ept * _NTILES - edges

    ch, nch, pad = _pick_ch(e)
    n_acc = math.ceil((n + 1) / (16 * _ZCH)) * (16 * _ZCH)

    if pad:
        src = jnp.concatenate(
            [edge_index[0], jnp.zeros((pad,), jnp.int32)])
        dst = jnp.concatenate(
            [edge_index[1], jnp.full((pad,), n, jnp.int32)])
    else:
        src, dst = edge_index[0], edge_index[1]
    src = src.reshape(_NTILES, nch, ch)
    dst = dst.reshape(_NTILES, nch, ch)

    z8 = jnp.zeros((_ZCH, 8), jnp.float32)
    o8 = jnp.ones((ch, 8), jnp.float32)
    h1_raw = _tc_mm(n, n_acc, f1)(features, W1)
    degp = _make_deg(n_acc, nch, ch, 4)(z8, o8, dst)
    h1s, dis = _tc_scale(n, n_acc, f1)(h1_raw, degp)
    q = _make_prop(n_acc, nch, ch, f1, 2)(h1s, src, dst)
    h2s = _tc_mid(n, n_acc, f2)(q, h1s, dis, b1.reshape(1, f1), W2)
    r = _make_prop(n_acc, nch, ch, f2, 4)(h2s, src, dst)
    h3s = _tc_mid(n, n_acc, f3)(r, h2s, dis, b2.reshape(1, f2), W3)
    sp = _make_prop(n_acc, nch, ch, f3, 4)(h3s, src, dst)
    abstract, pooled = _tc_final(n, f3)(sp, h3s, dis,
                                        b3.reshape(1, f3), att_W)
    return abstract, pooled.reshape(f3, 1)


# speedup vs baseline: 41.4606x; 1.0033x over previous
import functools
import math

import jax
import jax.numpy as jnp
from jax import lax
from jax.experimental import pallas as pl
from jax.experimental.pallas import tpu as pltpu
from jax.experimental.pallas import tpu_sc as plsc

_CH = 128
_ZCH = 128
_K = 2
_LANES = 16
_NTILES = 32


def _sc_mesh():
    return plsc.VectorSubcoreMesh(core_axis_name="c", subcore_axis_name="s")


def _make_deg(n_acc, nch, ch, k):
    w = 8
    zch = n_acc // (16 * _ZCH)

    @functools.partial(
        pl.kernel,
        out_type=jax.ShapeDtypeStruct((2, n_acc, w), jnp.float32),
        mesh=_sc_mesh(),
        compiler_params=pltpu.CompilerParams(use_tc_tiling_on_sc=False),
        scratch_types=[
            pltpu.VMEM((nch, ch), jnp.int32),
            pltpu.VMEM((ch, w), jnp.float32),
            pltpu.VMEM((_ZCH, w), jnp.float32),
            pltpu.VMEM_SHARED((n_acc, w), jnp.float32),
            pltpu.SemaphoreType.DMA,
        ],
    )
    def deg_kernel(zeros_hbm, ones_hbm, dst_hbm, out_hbm, dst_i, ones_v,
                   zero_v, acc_sh, sem):
        c = lax.axis_index("c")
        s = lax.axis_index("s")
        wid = s * 2 + c
        pltpu.sync_copy(dst_hbm.at[wid], dst_i)
        pltpu.sync_copy(zeros_hbm, zero_v)
        pltpu.sync_copy(ones_hbm, ones_v)

        def zacc(i, carry):
            pltpu.sync_copy(zero_v,
                            acc_sh.at[pl.ds((s * zch + i) * _ZCH, _ZCH)])
            return carry
        lax.fori_loop(0, zch, zacc, 0)
        plsc.subcore_barrier()

        def body(i, carry):
            for b in range(k):
                pltpu.async_copy(ones_v, acc_sh.at[dst_i.at[i * k + b]],
                                 sem, add=True)
            for b in range(k):
                pltpu.make_async_copy(
                    ones_v, acc_sh.at[dst_i.at[i * k + b]], sem).wait()
            return carry
        lax.fori_loop(0, nch // k, body, 0)
        plsc.subcore_barrier()

        rpt = n_acc // 16
        pltpu.sync_copy(acc_sh.at[pl.ds(s * rpt, rpt)],
                        out_hbm.at[c].at[pl.ds(s * rpt, rpt)])

    return deg_kernel


def _make_prop(n_acc, nch, ch, f, k):
    zch = n_acc // (16 * _ZCH)
    nb = nch // k

    @functools.partial(
        pl.kernel,
        out_type=jax.ShapeDtypeStruct((2, n_acc, f), jnp.float32),
        mesh=_sc_mesh(),
        compiler_params=pltpu.CompilerParams(use_tc_tiling_on_sc=False),
        scratch_types=[
            pltpu.VMEM((nch, ch), jnp.int32),
            pltpu.VMEM((k, ch), jnp.int32),
            pltpu.VMEM((k, ch), jnp.int32),
            pltpu.VMEM((k * ch, f), jnp.float32),
            pltpu.VMEM((k * ch, f), jnp.float32),
            pltpu.VMEM_SHARED((n_acc, f), jnp.float32),
            pltpu.VMEM_SHARED((n_acc, f), jnp.float32),
            pltpu.SemaphoreType.DMA,
            pltpu.SemaphoreType.DMA,
            pltpu.SemaphoreType.DMA,
            pltpu.SemaphoreType.DMA,
            pltpu.SemaphoreType.DMA,
        ],
    )
    def prop_kernel(h_hbm, src_hbm, dst_hbm, out_hbm,
                    src_i, di0, di1, rows0, rows1, acc_sh, tab_sh,
                    g0, g1, s0, s1, isem):
        c = lax.axis_index("c")
        s = lax.axis_index("s")
        wid = s * 2 + c
        pltpu.sync_copy(src_hbm.at[wid], src_i)
        rpt = n_acc // 16
        pltpu.sync_copy(h_hbm.at[pl.ds(s * rpt, rpt)],
                        tab_sh.at[pl.ds(s * rpt, rpt)])

        def zrow(i, carry):
            for j in range(f // _LANES):
                rows0[i, pl.ds(j * _LANES, _LANES)] = jnp.zeros(
                    (_LANES,), jnp.float32)
            return carry
        lax.fori_loop(0, _ZCH, zrow, 0)

        def zacc(i, carry):
            pltpu.sync_copy(rows0.at[pl.ds(0, _ZCH)],
                            acc_sh.at[pl.ds((s * zch + i) * _ZCH, _ZCH)])
            return carry
        lax.fori_loop(0, zch, zacc, 0)
        plsc.subcore_barrier()

        def idx_load(t, di):
            pltpu.sync_copy(dst_hbm.at[wid].at[pl.ds(t * k, k)], di)

        def fire(t, rows, sem):
            for b in range(k):
                pltpu.async_copy(tab_sh.at[src_i.at[t * k + b]],
                                 rows.at[pl.ds(b * ch, ch)], sem)

        def wait_rows(rows, sem):
            for b in range(k):
                pltpu.make_async_copy(tab_sh.at[src_i.at[0]],
                                      rows.at[pl.ds(b * ch, ch)],
                                      sem).wait()

        def scat(t, rows, di, sem):
            for b in range(k):
                pltpu.async_copy(rows.at[pl.ds(b * ch, ch)],
                                 acc_sh.at[di.at[b]], sem, add=True)

        def wait_scat(rows, di, sem):
            for b in range(k):
                pltpu.make_async_copy(rows.at[pl.ds(b * ch, ch)],
                                      acc_sh.at[di.at[b]], sem).wait()

        idx_load(0, di0)
        fire(0, rows0, g0)
        fire(1, rows1, g1)

        def body(i, carry):
            t = 2 * i
            pltpu.async_copy(dst_hbm.at[wid].at[pl.ds((t + 1) * k, k)],
                             di1, isem)
            wait_rows(rows0, g0)
            scat(t, rows0, di0, s0)
            wait_scat(rows0, di0, s0)

            @pl.when(t + 2 < nb)
            def _():
                fire(t + 2, rows0, g0)

            pltpu.make_async_copy(dst_hbm.at[wid].at[pl.ds(0, k)],
                                  di1, isem).wait()
            wait_rows(rows1, g1)
            scat(t + 1, rows1, di1, s1)
            wait_scat(rows1, di1, s1)

            @pl.when(t + 3 < nb)
            def _():
                fire(t + 3, rows1, g1)

            @pl.when(t + 2 < nb)
            def _():
                idx_load(t + 2, di0)
            return carry
        lax.fori_loop(0, nb // 2, body, 0)
        plsc.subcore_barrier()

        pltpu.sync_copy(acc_sh.at[pl.ds(s * rpt, rpt)],
                        out_hbm.at[c].at[pl.ds(s * rpt, rpt)])

    return prop_kernel


def _tc_first(n, n_acc, f1):
    def body(feat_ref, w_ref, degp_ref, h_ref, dis_ref):
        deg = degp_ref[0, 0:n, 0:1] + degp_ref[1, 0:n, 0:1] + 1.0
        dis = lax.rsqrt(deg)
        h = jnp.dot(feat_ref[...], w_ref[...],
                    precision=lax.Precision.DEFAULT,
                    preferred_element_type=jnp.float32)
        h_ref[0:n, :] = h * dis
        h_ref[n:n_acc, :] = jnp.zeros((n_acc - n, h.shape[1]), jnp.float32)
        dis_ref[...] = dis

    return pl.pallas_call(
        body,
        out_shape=(jax.ShapeDtypeStruct((n_acc, f1), jnp.float32),
                   jax.ShapeDtypeStruct((n, 1), jnp.float32)))


def _tc_mid(n, n_acc, f_out):
    def body(p_ref, h_ref, dis_ref, b_ref, w_ref, o_ref):
        dis = dis_ref[...]
        prop = p_ref[0, 0:n, :] + p_ref[1, 0:n, :] + h_ref[0:n, :]
        x = jnp.maximum(prop * dis + b_ref[...], 0.0)
        o_ref[0:n, :] = jnp.dot(x, w_ref[...],
                                precision=lax.Precision.DEFAULT,
                                preferred_element_type=jnp.float32) * dis
        o_ref[n:n_acc, :] = jnp.zeros((n_acc - n, w_ref.shape[1]),
                                      jnp.float32)

    return pl.pallas_call(
        body, out_shape=jax.ShapeDtypeStruct((n_acc, f_out), jnp.float32))


def _tc_final(n, f3):
    def body(p_ref, h_ref, dis_ref, b_ref, aw_ref, abs_ref, pool_ref):
        abstract = ((p_ref[0, 0:n, :] + p_ref[1, 0:n, :] + h_ref[0:n, :])
                    * dis_ref[...] + b_ref[...])
        abs_ref[...] = abstract
        gc = jnp.mean(jnp.dot(abstract, aw_ref[...],
                              precision=lax.Precision.DEFAULT,
                              preferred_element_type=jnp.float32),
                      axis=0, keepdims=True)
        tg = jnp.tanh(gc)
        scores = jax.nn.sigmoid(jnp.sum(abstract * tg, axis=1, keepdims=True))
        pool_ref[...] = jnp.sum(abstract * scores, axis=0, keepdims=True)

    return pl.pallas_call(
        body,
        out_shape=(jax.ShapeDtypeStruct((n, f3), jnp.float32),
                   jax.ShapeDtypeStruct((1, f3), jnp.float32)))


def kernel(edge_index, features, W1, b1, W2, b2, W3, b3, att_W):
    n, _ = features.shape
    e = edge_index.shape[1]
    f1, f2, f3 = W1.shape[1], W2.shape[1], W3.shape[1]

    def _pick_ch(edges):
        per_tile = edges // _NTILES
        if edges % _NTILES == 0:
            for c in range(_CH, 15, -1):
                if per_tile % (c * 8) == 0:
                    return c, per_tile // c, 0
        quantum = _CH * 8
        ept = math.ceil(edges / (_NTILES * quantum)) * quantum
        return _CH, ept // _CH, ept * _NTILES - edges

    ch, nch, pad = _pick_ch(e)
    n_acc = math.ceil((n + 1) / (16 * _ZCH)) * (16 * _ZCH)

    if pad:
        src = jnp.concatenate(
            [edge_index[0], jnp.zeros((pad,), jnp.int32)])
        dst = jnp.concatenate(
            [edge_index[1], jnp.full((pad,), n, jnp.int32)])
    else:
        src, dst = edge_index[0], edge_index[1]
    src = src.reshape(_NTILES, nch, ch)
    dst = dst.reshape(_NTILES, nch, ch)

    z8 = jnp.zeros((_ZCH, 8), jnp.float32)
    o8 = jnp.ones((ch, 8), jnp.float32)
    degp = _make_deg(n_acc, nch, ch, 4)(z8, o8, dst)
    h1s, dis = _tc_first(n, n_acc, f1)(features, W1, degp)
    q = _make_prop(n_acc, nch, ch, f1, 2)(h1s, src, dst)
    h2s = _tc_mid(n, n_acc, f2)(q, h1s, dis, b1.reshape(1, f1), W2)
    r = _make_prop(n_acc, nch, ch, f2, 4)(h2s, src, dst)
    h3s = _tc_mid(n, n_acc, f3)(r, h2s, dis, b2.reshape(1, f2), W3)
    sp = _make_prop(n_acc, nch, ch, f3, 4)(h3s, src, dst)
    abstract, pooled = _tc_final(n, f3)(sp, h3s, dis,
                                        b3.reshape(1, f3), att_W)
    return abstract, pooled.reshape(f3, 1)
